# baseline (speedup 1.0000x reference)
import functools

import jax
import jax.numpy as jnp
from jax import lax
from jax.experimental import pallas as pl
from jax.experimental.pallas import tpu as pltpu
from jax.experimental.pallas import tpu_sc as plsc

D = 32
L = 16
NW = 32
CHUNK = 128


@functools.lru_cache(maxsize=None)
def _build(B):
    BPW = B // NW
    NCH = BPW // CHUNK
    mesh = plsc.VectorSubcoreMesh(core_axis_name="c", subcore_axis_name="s")

    @functools.partial(
        pl.kernel,
        mesh=mesh,
        compiler_params=pltpu.CompilerParams(
            use_tc_tiling_on_sc=False, needs_layout_passes=False),
        out_type=jax.ShapeDtypeStruct((B,), jnp.float32),
        scratch_types=[
            pltpu.VMEM((NCH, CHUNK), jnp.int32),
            pltpu.VMEM((NCH, CHUNK), jnp.int32),
            pltpu.VMEM((BPW, D), jnp.float32),
            pltpu.VMEM((BPW, D), jnp.float32),
            pltpu.VMEM((BPW,), jnp.float32),
            pltpu.VMEM((BPW,), jnp.float32),
            pltpu.VMEM((L,), jnp.float32),
            pltpu.VMEM((BPW,), jnp.float32),
            pltpu.SemaphoreType.DMA,
        ],
    )
    def sc_kernel(ui_hbm, ii_hbm, ue_hbm, ie_hbm, ub_hbm, ib_hbm, gb_hbm,
                  out_hbm, ui_v, ii_v, ue_v, ie_v, ub_v, ib_v, gb_v, pred_v,
                  sem):
        wid = lax.axis_index("s") * 2 + lax.axis_index("c")
        base = wid * BPW

        pltpu.sync_copy(ui_hbm.at[pl.ds(wid * NCH, NCH)], ui_v)
        pltpu.sync_copy(ii_hbm.at[pl.ds(wid * NCH, NCH)], ii_v)
        pltpu.sync_copy(gb_hbm, gb_v)

        copies = []
        for j in range(NCH):
            sl = pl.ds(j * CHUNK, CHUNK)
            copies.append(pltpu.async_copy(ue_hbm.at[ui_v.at[j]], ue_v.at[sl], sem))
            copies.append(pltpu.async_copy(ie_hbm.at[ii_v.at[j]], ie_v.at[sl], sem))
            copies.append(pltpu.async_copy(ub_hbm.at[ui_v.at[j]], ub_v.at[sl], sem))
            copies.append(pltpu.async_copy(ib_hbm.at[ii_v.at[j]], ib_v.at[sl], sem))
        for c in copies:
            c.wait()

        gbs = gb_v[...]

        def body(g, carry):
            row0 = g * L
            riota = lax.iota(jnp.int32, L) + row0
            acc = jnp.zeros((L,), jnp.float32)
            for j in range(D):
                cj = jnp.full((L,), j, jnp.int32)
                u = plsc.load_gather(ue_v, [riota, cj])
                t = plsc.load_gather(ie_v, [riota, cj])
                acc = acc + jnp.maximum(u, 0.0) * jnp.maximum(t, 0.0)
            ub = ub_v[pl.ds(row0, L)]
            ib = ib_v[pl.ds(row0, L)]
            pred_v[pl.ds(row0, L)] = acc + ub + ib + gbs
            return carry

        lax.fori_loop(0, BPW // L, body, 0)
        pltpu.sync_copy(pred_v, out_hbm.at[pl.ds(base, BPW)])

    return sc_kernel


def kernel(user_indices, item_indices, user_emb, item_emb, user_bias,
           item_bias, global_bias):
    B = user_indices.shape[0]
    ui = user_indices.reshape(B // CHUNK, CHUNK)
    ii = item_indices.reshape(B // CHUNK, CHUNK)
    gb = jnp.broadcast_to(global_bias, (L,))
    return _build(B)(ui, ii, user_emb, item_emb, user_bias.reshape(-1),
                     item_bias.reshape(-1), gb)
